# baseline (speedup 1.0000x reference)
import functools
import math

import jax
import jax.numpy as jnp
from jax import lax
from jax.experimental import pallas as pl
from jax.experimental.pallas import tpu as pltpu
from jax.experimental.pallas import tpu_sc as plsc

N = 10000
NP = 10240
E = 320000
EPS = 1e-5
S_BN = 1.0 / math.sqrt(1.0 + EPS)

NC, NS = 2, 16
NW = NC * NS
CE = 3200
CED = 2000

BN = 2048
NB = NP // BN


def _mesh():
    return plsc.VectorSubcoreMesh(
        core_axis_name="c", subcore_axis_name="s", num_cores=NC, num_subcores=NS
    )


def _deg_body(row_h, out_h, deg_v, rowb_v):
    c = lax.axis_index("c")
    s = lax.axis_index("s")
    wid = s * NC + c
    ept = E // NW
    base = wid * ept
    zeros = jnp.zeros((16,), jnp.float32)

    def zero_b(i, _):
        deg_v[pl.ds(i * 16, 16)] = zeros
        return 0

    lax.fori_loop(0, NP // 16, zero_b, 0)

    ones = jnp.ones((16,), jnp.float32)
    grp = CED // 16

    def chunk_b(i, _):
        pltpu.sync_copy(row_h.at[pl.ds(base + i * CED, CED)], rowb_v)

        @plsc.parallel_loop(0, grp, 1, unroll=2)
        def grp_b(g):
            r = rowb_v[pl.ds(g * 16, 16)]
            plsc.addupdate_scatter(deg_v, [r], ones)

        return 0

    lax.fori_loop(0, ept // CED, chunk_b, 0)
    pltpu.sync_copy(deg_v, out_h.at[wid])


_SC_PARAMS = pltpu.CompilerParams(needs_layout_passes=False)

_deg_call = pl.kernel(
    _deg_body,
    out_type=jax.ShapeDtypeStruct((NW, NP), jnp.float32),
    mesh=_mesh(),
    compiler_params=_SC_PARAMS,
    scratch_types=[
        pltpu.VMEM((NP,), jnp.float32),
        pltpu.VMEM((CED,), jnp.int32),
    ],
)


def _conv_body(C, z_h, ei_h, ew_h, out_h, zb, acc,
               rcb0, rcb1, ewb0, ewb1, zsem, rs0, rs1, es0, es1):
    F = C // NS
    FNP = F * NP
    c = lax.axis_index("c")
    s = lax.axis_index("s")
    ehalf = E // NC
    base = c * ehalf
    nch = ehalf // CE

    zdesc = pltpu.async_copy(z_h.at[pl.ds(s * FNP, FNP)], zb, zsem)

    zeros = jnp.zeros((16,), jnp.float32)

    @plsc.parallel_loop(0, FNP // 16, 1, unroll=8)
    def zero_b(i):
        acc[pl.ds(i * 16, 16)] = zeros

    zdesc.wait()

    def start(rcb_s, ewb_s, rs, es, i):
        eb = base + i * CE
        pltpu.async_copy(ei_h.at[:, pl.ds(eb, CE)], rcb_s, rs)
        pltpu.async_copy(ew_h.at[pl.ds(eb, CE)], ewb_s, es)

    def wait(rcb_s, ewb_s, rs, es):
        pltpu.make_async_copy(ei_h.at[:, pl.ds(base, CE)], rcb_s, rs).wait()
        pltpu.make_async_copy(ew_h.at[pl.ds(base, CE)], ewb_s, es).wait()

    grp = CE // 16

    def process(rcb_s, ewb_s):
        @plsc.parallel_loop(0, grp, 1, unroll=4)
        def grp_b(g):
            off = g * 16
            r = rcb_s[0, pl.ds(off, 16)]
            ci = rcb_s[1, pl.ds(off, 16)]
            w = ewb_s[pl.ds(off, 16)]
            for f in range(F):
                v = plsc.load_gather(zb.at[pl.ds(f * NP, NP)], [r]) * w
                plsc.addupdate_scatter(acc.at[pl.ds(f * NP, NP)], [ci], v)

    start(rcb0, ewb0, rs0, es0, 0)
    start(rcb1, ewb1, rs1, es1, 1)

    def pair_b(k, _):
        i0 = 2 * k
        wait(rcb0, ewb0, rs0, es0)
        process(rcb0, ewb0)

        @pl.when(i0 + 2 < nch)
        def _n0():
            start(rcb0, ewb0, rs0, es0, i0 + 2)

        wait(rcb1, ewb1, rs1, es1)
        process(rcb1, ewb1)

        @pl.when(i0 + 3 < nch)
        def _n1():
            start(rcb1, ewb1, rs1, es1, i0 + 3)

        return 0

    lax.fori_loop(0, nch // 2, pair_b, 0)
    pltpu.sync_copy(acc, out_h.at[pl.ds(c * (C * NP) + s * FNP, FNP)])


def _make_conv(C):
    F = C // NS
    return pl.kernel(
        functools.partial(_conv_body, C),
        out_type=jax.ShapeDtypeStruct((NC * C * NP,), jnp.float32),
        mesh=_mesh(),
        compiler_params=_SC_PARAMS,
        scratch_types=[
            pltpu.VMEM((F * NP,), jnp.float32),
            pltpu.VMEM((F * NP,), jnp.float32),
            pltpu.VMEM((2, CE), jnp.int32),
            pltpu.VMEM((2, CE), jnp.int32),
            pltpu.VMEM((CE,), jnp.float32),
            pltpu.VMEM((CE,), jnp.float32),
            pltpu.SemaphoreType.DMA,
            pltpu.SemaphoreType.DMA,
            pltpu.SemaphoreType.DMA,
            pltpu.SemaphoreType.DMA,
            pltpu.SemaphoreType.DMA,
        ],
    )


_conv64 = _make_conv(64)
_conv32 = _make_conv(32)


def _prep_body(degp, xT, W0, g0, dinv_o, z0_o):
    deg = jnp.sum(degp[...], axis=0, keepdims=True)
    dinv = jnp.where(deg > 0, lax.rsqrt(jnp.maximum(deg, 1.0)), 0.0)
    dinv_o[...] = dinv
    Wf = (S_BN * g0[...]) * W0[...]
    mm = lax.dot_general(Wf, xT[...], (((1,), (0,)), ((), ())),
                         preferred_element_type=jnp.float32)
    z0_o[...] = mm * dinv


def _full(shape):
    return pl.BlockSpec(shape, lambda i: tuple(0 for _ in shape))


_prep_call = pl.pallas_call(
    _prep_body,
    grid=(NB,),
    in_specs=[
        pl.BlockSpec((NW, BN), lambda i: (0, i)),
        pl.BlockSpec((128, BN), lambda i: (0, i)),
        _full((64, 128)),
        _full((64, 1)),
    ],
    out_specs=[
        pl.BlockSpec((1, BN), lambda i: (0, i)),
        pl.BlockSpec((64, BN), lambda i: (0, i)),
    ],
    out_shape=[
        jax.ShapeDtypeStruct((1, NP), jnp.float32),
        jax.ShapeDtypeStruct((64, NP), jnp.float32),
    ],
)


def _mid_body(scale_next, P, dinv, g, b, be, Wn, gn, z_o):
    Pb = P[...]
    ps = Pb[0] + Pb[1]
    dv = dinv[...]
    bf = S_BN * g[...] * b[...] + be[...]
    X = jnp.maximum(dv * ps + bf, 0.0)
    if scale_next:
        Wf = (S_BN * gn[...]) * Wn[...]
    else:
        Wf = Wn[...]
    z_o[...] = lax.dot_general(Wf, X, (((1,), (0,)), ((), ())),
                               preferred_element_type=jnp.float32) * dv


def _make_mid(C, C2, scale_next):
    return pl.pallas_call(
        functools.partial(_mid_body, scale_next),
        grid=(NB,),
        in_specs=[
            pl.BlockSpec((NC, C, BN), lambda i: (0, 0, i)),
            pl.BlockSpec((1, BN), lambda i: (0, i)),
            _full((C, 1)),
            _full((C, 1)),
            _full((C, 1)),
            _full((C2, C)),
            _full((C2, 1)),
        ],
        out_specs=pl.BlockSpec((C2, BN), lambda i: (0, i)),
        out_shape=jax.ShapeDtypeStruct((C2, NP), jnp.float32),
    )


_mid1 = _make_mid(64, 64, True)
_mid2 = _make_mid(64, 32, False)


def _final_body(P, dinv, b2, s_o):
    Pb = P[...]
    s_o[...] = dinv[...] * (Pb[0] + Pb[1]) + b2[...]


_final_call = pl.pallas_call(
    _final_body,
    grid=(NB,),
    in_specs=[
        pl.BlockSpec((NC, 32, BN), lambda i: (0, 0, i)),
        pl.BlockSpec((1, BN), lambda i: (0, i)),
        _full((32, 1)),
    ],
    out_specs=pl.BlockSpec((32, BN), lambda i: (0, i)),
    out_shape=jax.ShapeDtypeStruct((32, NP), jnp.float32),
)


def kernel(node_features, edge_index, edge_weights, W0, b0, g0, be0,
           W1, b1, g1, be1, W2, b2):
    ei = edge_index.astype(jnp.int32)
    ew = edge_weights
    xT = jnp.pad(node_features.T, ((0, 0), (0, NP - N)))

    degp = _deg_call(ei[0])
    dinv, Z0 = _prep_call(degp, xT, W0, g0.reshape(64, 1))

    P = _conv64(Z0.reshape(-1), ei, ew).reshape(NC, 64, NP)
    Z1 = _mid1(P, dinv, g0.reshape(64, 1), b0.reshape(64, 1),
               be0.reshape(64, 1), W1, g1.reshape(64, 1))
    P = _conv64(Z1.reshape(-1), ei, ew).reshape(NC, 64, NP)
    Z2 = _mid2(P, dinv, g1.reshape(64, 1), b1.reshape(64, 1),
               be1.reshape(64, 1), W2, g1.reshape(64, 1))
    P = _conv32(Z2.reshape(-1), ei, ew).reshape(NC, 32, NP)
    S = _final_call(P, dinv, b2.reshape(32, 1))
    return S.T[:N]


# speedup vs baseline: 1.0796x; 1.0796x over previous
import functools
import math

import jax
import jax.numpy as jnp
from jax import lax
from jax.experimental import pallas as pl
from jax.experimental.pallas import tpu as pltpu
from jax.experimental.pallas import tpu_sc as plsc

N = 10000
NP = 10240
E = 320000
EPS = 1e-5
S_BN = 1.0 / math.sqrt(1.0 + EPS)

NC, NS = 2, 16
NW = NC * NS
CE = 3200
CED = 2000

BN = 2048
NB = NP // BN


def _mesh():
    return plsc.VectorSubcoreMesh(
        core_axis_name="c", subcore_axis_name="s", num_cores=NC, num_subcores=NS
    )


def _deg_body(eif_h, out_h, rcp_h, deg_v, rowb_v, colb_v, rcpb_v):
    c = lax.axis_index("c")
    s = lax.axis_index("s")
    wid = s * NC + c
    ept = E // NW
    base = wid * ept
    zeros = jnp.zeros((16,), jnp.float32)

    def zero_b(i, _):
        deg_v[pl.ds(i * 16, 16)] = zeros
        return 0

    lax.fori_loop(0, NP // 16, zero_b, 0)

    ones = jnp.ones((16,), jnp.float32)
    grp = CED // 16

    def chunk_b(i, _):
        eb = base + i * CED
        pltpu.sync_copy(eif_h.at[pl.ds(eb, CED)], rowb_v)
        pltpu.sync_copy(eif_h.at[pl.ds(E + eb, CED)], colb_v)

        @plsc.parallel_loop(0, grp, 1, unroll=2)
        def grp_b(g):
            off = g * 16
            r = rowb_v[pl.ds(off, 16)]
            ci = colb_v[pl.ds(off, 16)]
            rcpb_v[pl.ds(off, 16)] = r | (ci << 14)
            plsc.addupdate_scatter(deg_v, [r], ones)

        pltpu.sync_copy(rcpb_v, rcp_h.at[pl.ds(eb, CED)])
        return 0

    lax.fori_loop(0, ept // CED, chunk_b, 0)
    pltpu.sync_copy(deg_v, out_h.at[wid])


_SC_PARAMS = pltpu.CompilerParams(needs_layout_passes=False)

_deg_call = pl.kernel(
    _deg_body,
    out_type=[
        jax.ShapeDtypeStruct((NW, NP), jnp.float32),
        jax.ShapeDtypeStruct((E,), jnp.int32),
    ],
    mesh=_mesh(),
    compiler_params=_SC_PARAMS,
    scratch_types=[
        pltpu.VMEM((NP,), jnp.float32),
        pltpu.VMEM((CED,), jnp.int32),
        pltpu.VMEM((CED,), jnp.int32),
        pltpu.VMEM((CED,), jnp.int32),
    ],
)


def _conv_body(C, z_h, rcp_h, ew_h, out_h, zb, acc,
               rcb0, rcb1, ewb0, ewb1, zsem, rs0, rs1, es0, es1):
    F = C // NS
    FNP = F * NP
    c = lax.axis_index("c")
    s = lax.axis_index("s")
    ehalf = E // NC
    base = c * ehalf
    nch = ehalf // CE

    zdesc = pltpu.async_copy(z_h.at[pl.ds(s * FNP, FNP)], zb, zsem)

    zeros = jnp.zeros((16,), jnp.float32)

    @plsc.parallel_loop(0, FNP // 16, 1, unroll=8)
    def zero_b(i):
        acc[pl.ds(i * 16, 16)] = zeros

    zdesc.wait()

    def start(rcb_s, ewb_s, rs, es, i):
        eb = base + i * CE
        pltpu.async_copy(rcp_h.at[pl.ds(eb, CE)], rcb_s, rs)
        pltpu.async_copy(ew_h.at[pl.ds(eb, CE)], ewb_s, es)

    def wait(rcb_s, ewb_s, rs, es):
        pltpu.make_async_copy(rcp_h.at[pl.ds(base, CE)], rcb_s, rs).wait()
        pltpu.make_async_copy(ew_h.at[pl.ds(base, CE)], ewb_s, es).wait()

    grp = CE // 16

    def process(rcb_s, ewb_s):
        @plsc.parallel_loop(0, grp, 1, unroll=2)
        def grp_b(g):
            off = g * 16
            rc = rcb_s[pl.ds(off, 16)]
            r = rc & 0x3FFF
            ci = lax.shift_right_logical(rc, 14)
            w = ewb_s[pl.ds(off, 16)]
            for f in range(F):
                v = plsc.load_gather(zb.at[pl.ds(f * NP, NP)], [r]) * w
                plsc.addupdate_scatter(acc.at[pl.ds(f * NP, NP)], [ci], v)

    start(rcb0, ewb0, rs0, es0, 0)
    start(rcb1, ewb1, rs1, es1, 1)

    def pair_b(k, _):
        i0 = 2 * k
        wait(rcb0, ewb0, rs0, es0)
        process(rcb0, ewb0)

        @pl.when(i0 + 2 < nch)
        def _n0():
            start(rcb0, ewb0, rs0, es0, i0 + 2)

        wait(rcb1, ewb1, rs1, es1)
        process(rcb1, ewb1)

        @pl.when(i0 + 3 < nch)
        def _n1():
            start(rcb1, ewb1, rs1, es1, i0 + 3)

        return 0

    lax.fori_loop(0, nch // 2, pair_b, 0)
    pltpu.sync_copy(acc, out_h.at[pl.ds(c * (C * NP) + s * FNP, FNP)])


def _make_conv(C):
    F = C // NS
    return pl.kernel(
        functools.partial(_conv_body, C),
        out_type=jax.ShapeDtypeStruct((NC * C * NP,), jnp.float32),
        mesh=_mesh(),
        compiler_params=_SC_PARAMS,
        scratch_types=[
            pltpu.VMEM((F * NP,), jnp.float32),
            pltpu.VMEM((F * NP,), jnp.float32),
            pltpu.VMEM((CE,), jnp.int32),
            pltpu.VMEM((CE,), jnp.int32),
            pltpu.VMEM((CE,), jnp.float32),
            pltpu.VMEM((CE,), jnp.float32),
            pltpu.SemaphoreType.DMA,
            pltpu.SemaphoreType.DMA,
            pltpu.SemaphoreType.DMA,
            pltpu.SemaphoreType.DMA,
            pltpu.SemaphoreType.DMA,
        ],
    )


_conv64 = _make_conv(64)
_conv32 = _make_conv(32)


def _prep_body(degp, xT, W0, g0, dinv_o, z0_o):
    deg = jnp.sum(degp[...], axis=0, keepdims=True)
    dinv = jnp.where(deg > 0, lax.rsqrt(jnp.maximum(deg, 1.0)), 0.0)
    dinv_o[...] = dinv
    Wf = (S_BN * g0[...]) * W0[...]
    mm = lax.dot_general(Wf, xT[...], (((1,), (0,)), ((), ())),
                         preferred_element_type=jnp.float32)
    z0_o[...] = mm * dinv


def _full(shape):
    return pl.BlockSpec(shape, lambda i: tuple(0 for _ in shape))


_prep_call = pl.pallas_call(
    _prep_body,
    grid=(NB,),
    in_specs=[
        pl.BlockSpec((NW, BN), lambda i: (0, i)),
        pl.BlockSpec((128, BN), lambda i: (0, i)),
        _full((64, 128)),
        _full((64, 1)),
    ],
    out_specs=[
        pl.BlockSpec((1, BN), lambda i: (0, i)),
        pl.BlockSpec((64, BN), lambda i: (0, i)),
    ],
    out_shape=[
        jax.ShapeDtypeStruct((1, NP), jnp.float32),
        jax.ShapeDtypeStruct((64, NP), jnp.float32),
    ],
)


def _mid_body(scale_next, P, dinv, g, b, be, Wn, gn, z_o):
    Pb = P[...]
    ps = Pb[0] + Pb[1]
    dv = dinv[...]
    bf = S_BN * g[...] * b[...] + be[...]
    X = jnp.maximum(dv * ps + bf, 0.0)
    if scale_next:
        Wf = (S_BN * gn[...]) * Wn[...]
    else:
        Wf = Wn[...]
    z_o[...] = lax.dot_general(Wf, X, (((1,), (0,)), ((), ())),
                               preferred_element_type=jnp.float32) * dv


def _make_mid(C, C2, scale_next):
    return pl.pallas_call(
        functools.partial(_mid_body, scale_next),
        grid=(NB,),
        in_specs=[
            pl.BlockSpec((NC, C, BN), lambda i: (0, 0, i)),
            pl.BlockSpec((1, BN), lambda i: (0, i)),
            _full((C, 1)),
            _full((C, 1)),
            _full((C, 1)),
            _full((C2, C)),
            _full((C2, 1)),
        ],
        out_specs=pl.BlockSpec((C2, BN), lambda i: (0, i)),
        out_shape=jax.ShapeDtypeStruct((C2, NP), jnp.float32),
    )


_mid1 = _make_mid(64, 64, True)
_mid2 = _make_mid(64, 32, False)


def _final_body(P, dinv, b2, s_o):
    Pb = P[...]
    s_o[...] = dinv[...] * (Pb[0] + Pb[1]) + b2[...]


_final_call = pl.pallas_call(
    _final_body,
    grid=(NB,),
    in_specs=[
        pl.BlockSpec((NC, 32, BN), lambda i: (0, 0, i)),
        pl.BlockSpec((1, BN), lambda i: (0, i)),
        _full((32, 1)),
    ],
    out_specs=pl.BlockSpec((32, BN), lambda i: (0, i)),
    out_shape=jax.ShapeDtypeStruct((32, NP), jnp.float32),
)


def kernel(node_features, edge_index, edge_weights, W0, b0, g0, be0,
           W1, b1, g1, be1, W2, b2):
    ei = edge_index.astype(jnp.int32)
    ew = edge_weights
    xT = jnp.pad(node_features.T, ((0, 0), (0, NP - N)))

    degp, rcp = _deg_call(ei.reshape(-1))
    dinv, Z0 = _prep_call(degp, xT, W0, g0.reshape(64, 1))

    P = _conv64(Z0.reshape(-1), rcp, ew).reshape(NC, 64, NP)
    Z1 = _mid1(P, dinv, g0.reshape(64, 1), b0.reshape(64, 1),
               be0.reshape(64, 1), W1, g1.reshape(64, 1))
    P = _conv64(Z1.reshape(-1), rcp, ew).reshape(NC, 64, NP)
    Z2 = _mid2(P, dinv, g1.reshape(64, 1), b1.reshape(64, 1),
               be1.reshape(64, 1), W2, g1.reshape(64, 1))
    P = _conv32(Z2.reshape(-1), rcp, ew).reshape(NC, 32, NP)
    S = _final_call(P, dinv, b2.reshape(32, 1))
    return S.T[:N]


# speedup vs baseline: 1.2010x; 1.1125x over previous
import functools
import math

import jax
import jax.numpy as jnp
from jax import lax
from jax.experimental import pallas as pl
from jax.experimental.pallas import tpu as pltpu
from jax.experimental.pallas import tpu_sc as plsc

N = 10000
NP = 10240
E = 320000
EPS = 1e-5
S_BN = 1.0 / math.sqrt(1.0 + EPS)

NC, NS = 2, 16
NW = NC * NS
CE = 3200
CED = 2000

BN = 2048
NB = NP // BN


def _mesh():
    return plsc.VectorSubcoreMesh(
        core_axis_name="c", subcore_axis_name="s", num_cores=NC, num_subcores=NS
    )


def _deg_body(eif_h, out_h, rcp_h, deg_v, rowb_v, colb_v, rcpb_v):
    c = lax.axis_index("c")
    s = lax.axis_index("s")
    wid = s * NC + c
    ept = E // NW
    base = wid * ept
    zeros = jnp.zeros((16,), jnp.float32)

    def zero_b(i, _):
        deg_v[pl.ds(i * 16, 16)] = zeros
        return 0

    lax.fori_loop(0, NP // 16, zero_b, 0)

    ones = jnp.ones((16,), jnp.float32)
    grp = CED // 16

    def chunk_b(i, _):
        eb = base + i * CED
        pltpu.sync_copy(eif_h.at[pl.ds(eb, CED)], rowb_v)
        pltpu.sync_copy(eif_h.at[pl.ds(E + eb, CED)], colb_v)

        @plsc.parallel_loop(0, grp, 1, unroll=2)
        def grp_b(g):
            off = g * 16
            r = rowb_v[pl.ds(off, 16)]
            ci = colb_v[pl.ds(off, 16)]
            rcpb_v[pl.ds(off, 16)] = r | (ci << 14)
            plsc.addupdate_scatter(deg_v, [r], ones)

        pltpu.sync_copy(rcpb_v, rcp_h.at[pl.ds(eb, CED)])
        return 0

    lax.fori_loop(0, ept // CED, chunk_b, 0)
    pltpu.sync_copy(deg_v, out_h.at[wid])


_SC_PARAMS = pltpu.CompilerParams(needs_layout_passes=False)

_deg_call = pl.kernel(
    _deg_body,
    out_type=[
        jax.ShapeDtypeStruct((NW, NP), jnp.float32),
        jax.ShapeDtypeStruct((E,), jnp.int32),
    ],
    mesh=_mesh(),
    compiler_params=_SC_PARAMS,
    scratch_types=[
        pltpu.VMEM((NP,), jnp.float32),
        pltpu.VMEM((CED,), jnp.int32),
        pltpu.VMEM((CED,), jnp.int32),
        pltpu.VMEM((CED,), jnp.int32),
    ],
)


def _conv_body(C, z_h, rcp_h, ew_h, out_h, zb, acc,
               rcb0, rcb1, ewb0, ewb1, zsem, rs0, rs1, es0, es1):
    F = C // NS
    F2 = F // 2
    half = C // 2
    FNP = F * NP
    c = lax.axis_index("c")
    s = lax.axis_index("s")
    ehalf = E // NC
    base = c * ehalf
    nch = ehalf // CE

    zdesc = pltpu.async_copy(z_h.at[pl.ds(s * F2 * NP, F2 * NP)], zb, zsem)

    zeros = jnp.zeros((16,), jnp.float32)

    @plsc.parallel_loop(0, FNP // 16, 1, unroll=8)
    def zero_b(i):
        acc[pl.ds(i * 16, 16)] = zeros

    zdesc.wait()

    def start(rcb_s, ewb_s, rs, es, i):
        eb = base + i * CE
        pltpu.async_copy(rcp_h.at[pl.ds(eb, CE)], rcb_s, rs)
        pltpu.async_copy(ew_h.at[pl.ds(eb, CE)], ewb_s, es)

    def wait(rcb_s, ewb_s, rs, es):
        pltpu.make_async_copy(rcp_h.at[pl.ds(base, CE)], rcb_s, rs).wait()
        pltpu.make_async_copy(ew_h.at[pl.ds(base, CE)], ewb_s, es).wait()

    grp = CE // 16

    def process(rcb_s, ewb_s):
        @plsc.parallel_loop(0, grp, 1, unroll=2)
        def grp_b(g):
            off = g * 16
            rc = rcb_s[pl.ds(off, 16)]
            r = rc & 0x3FFF
            ci = lax.shift_right_logical(rc, 14)
            w = ewb_s[pl.ds(off, 16)]
            for j in range(F2):
                vp = plsc.load_gather(zb.at[pl.ds(j * NP, NP)], [r])
                va = plsc.bitcast(jnp.left_shift(vp, 16), jnp.float32) * w
                vb = plsc.bitcast(vp & jnp.int32(-65536), jnp.float32) * w
                plsc.addupdate_scatter(acc.at[pl.ds(j * NP, NP)], [ci], va)
                plsc.addupdate_scatter(acc.at[pl.ds((F2 + j) * NP, NP)], [ci], vb)

    start(rcb0, ewb0, rs0, es0, 0)
    start(rcb1, ewb1, rs1, es1, 1)

    def pair_b(k, _):
        i0 = 2 * k
        wait(rcb0, ewb0, rs0, es0)
        process(rcb0, ewb0)

        @pl.when(i0 + 2 < nch)
        def _n0():
            start(rcb0, ewb0, rs0, es0, i0 + 2)

        wait(rcb1, ewb1, rs1, es1)
        process(rcb1, ewb1)

        @pl.when(i0 + 3 < nch)
        def _n1():
            start(rcb1, ewb1, rs1, es1, i0 + 3)

        return 0

    lax.fori_loop(0, nch // 2, pair_b, 0)
    pltpu.sync_copy(acc.at[pl.ds(0, F2 * NP)],
                    out_h.at[pl.ds(c * (C * NP) + s * F2 * NP, F2 * NP)])
    pltpu.sync_copy(acc.at[pl.ds(F2 * NP, F2 * NP)],
                    out_h.at[pl.ds(c * (C * NP) + (half + s * F2) * NP, F2 * NP)])


def _make_conv(C):
    F = C // NS
    return pl.kernel(
        functools.partial(_conv_body, C),
        out_type=jax.ShapeDtypeStruct((NC * C * NP,), jnp.float32),
        mesh=_mesh(),
        compiler_params=_SC_PARAMS,
        scratch_types=[
            pltpu.VMEM((C // NS // 2 * NP,), jnp.int32),
            pltpu.VMEM((F * NP,), jnp.float32),
            pltpu.VMEM((CE,), jnp.int32),
            pltpu.VMEM((CE,), jnp.int32),
            pltpu.VMEM((CE,), jnp.float32),
            pltpu.VMEM((CE,), jnp.float32),
            pltpu.SemaphoreType.DMA,
            pltpu.SemaphoreType.DMA,
            pltpu.SemaphoreType.DMA,
            pltpu.SemaphoreType.DMA,
            pltpu.SemaphoreType.DMA,
        ],
    )


_conv64 = _make_conv(64)
_conv32 = _make_conv(32)


def _pack_pairs(z):
    half = z.shape[0] // 2
    au = lax.bitcast_convert_type(
        z[:half].astype(jnp.bfloat16), jnp.uint16).astype(jnp.uint32)
    bu = lax.bitcast_convert_type(
        z[half:].astype(jnp.bfloat16), jnp.uint16).astype(jnp.uint32)
    return lax.bitcast_convert_type(au | (bu << 16), jnp.int32)


def _prep_body(degp, xT, W0, g0, dinv_o, z0_o):
    deg = jnp.sum(degp[...], axis=0, keepdims=True)
    dinv = jnp.where(deg > 0, lax.rsqrt(jnp.maximum(deg, 1.0)), 0.0)
    dinv_o[...] = dinv
    Wf = (S_BN * g0[...]) * W0[...]
    mm = lax.dot_general(Wf, xT[...], (((1,), (0,)), ((), ())),
                         preferred_element_type=jnp.float32)
    z0_o[...] = _pack_pairs(mm * dinv)


def _full(shape):
    return pl.BlockSpec(shape, lambda i: tuple(0 for _ in shape))


_prep_call = pl.pallas_call(
    _prep_body,
    grid=(NB,),
    in_specs=[
        pl.BlockSpec((NW, BN), lambda i: (0, i)),
        pl.BlockSpec((128, BN), lambda i: (0, i)),
        _full((64, 128)),
        _full((64, 1)),
    ],
    out_specs=[
        pl.BlockSpec((1, BN), lambda i: (0, i)),
        pl.BlockSpec((32, BN), lambda i: (0, i)),
    ],
    out_shape=[
        jax.ShapeDtypeStruct((1, NP), jnp.float32),
        jax.ShapeDtypeStruct((32, NP), jnp.int32),
    ],
)


def _mid_body(scale_next, P, dinv, g, b, be, Wn, gn, z_o):
    Pb = P[...]
    ps = Pb[0] + Pb[1]
    dv = dinv[...]
    bf = S_BN * g[...] * b[...] + be[...]
    X = jnp.maximum(dv * ps + bf, 0.0)
    if scale_next:
        Wf = (S_BN * gn[...]) * Wn[...]
    else:
        Wf = Wn[...]
    mm = lax.dot_general(Wf, X, (((1,), (0,)), ((), ())),
                         preferred_element_type=jnp.float32)
    z_o[...] = _pack_pairs(mm * dv)


def _make_mid(C, C2, scale_next):
    return pl.pallas_call(
        functools.partial(_mid_body, scale_next),
        grid=(NB,),
        in_specs=[
            pl.BlockSpec((NC, C, BN), lambda i: (0, 0, i)),
            pl.BlockSpec((1, BN), lambda i: (0, i)),
            _full((C, 1)),
            _full((C, 1)),
            _full((C, 1)),
            _full((C2, C)),
            _full((C2, 1)),
        ],
        out_specs=pl.BlockSpec((C2 // 2, BN), lambda i: (0, i)),
        out_shape=jax.ShapeDtypeStruct((C2 // 2, NP), jnp.int32),
    )


_mid1 = _make_mid(64, 64, True)
_mid2 = _make_mid(64, 32, False)


def _final_body(P, dinv, b2, s_o):
    Pb = P[...]
    s_o[...] = dinv[...] * (Pb[0] + Pb[1]) + b2[...]


_final_call = pl.pallas_call(
    _final_body,
    grid=(NB,),
    in_specs=[
        pl.BlockSpec((NC, 32, BN), lambda i: (0, 0, i)),
        pl.BlockSpec((1, BN), lambda i: (0, i)),
        _full((32, 1)),
    ],
    out_specs=pl.BlockSpec((32, BN), lambda i: (0, i)),
    out_shape=jax.ShapeDtypeStruct((32, NP), jnp.float32),
)


def kernel(node_features, edge_index, edge_weights, W0, b0, g0, be0,
           W1, b1, g1, be1, W2, b2):
    ei = edge_index.astype(jnp.int32)
    ew = edge_weights
    xT = jnp.pad(node_features.T, ((0, 0), (0, NP - N)))

    degp, rcp = _deg_call(ei.reshape(-1))
    dinv, Z0 = _prep_call(degp, xT, W0, g0.reshape(64, 1))

    P = _conv64(Z0.reshape(-1), rcp, ew).reshape(NC, 64, NP)
    Z1 = _mid1(P, dinv, g0.reshape(64, 1), b0.reshape(64, 1),
               be0.reshape(64, 1), W1, g1.reshape(64, 1))
    P = _conv64(Z1.reshape(-1), rcp, ew).reshape(NC, 64, NP)
    Z2 = _mid2(P, dinv, g1.reshape(64, 1), b1.reshape(64, 1),
               be1.reshape(64, 1), W2, g1.reshape(64, 1))
    P = _conv32(Z2.reshape(-1), rcp, ew).reshape(NC, 32, NP)
    S = _final_call(P, dinv, b2.reshape(32, 1))
    return S.T[:N]
